# Initial kernel scaffold; baseline (speedup 1.0000x reference)
import jax
import jax.numpy as jnp
from jax.experimental import pallas as pl


def kernel(ray_o, ray_d, pts):
    raise NotImplementedError("write your pallas kernel here")


# speedup vs baseline: 61.9717x; 61.9717x over previous
import math

import jax
import jax.numpy as jnp
from jax import lax
from jax.experimental import pallas as pl
from jax.experimental.pallas import tpu as pltpu
from jax.experimental.pallas import tpu_sc as plsc

KSEL = 8
NRAY = 8
NF = 512
NPTS = 10000
NPAD = 10240
BF = 8
NSEL = NF * NRAY * KSEL
DPAD = 16


def _dist_topk_body(ro_ref, rd_ref, pts_ref, dist_ref, idx_ref):
    px = pts_ref[0:1, :]
    py = pts_ref[1:2, :]
    pz = pts_ref[2:3, :]
    ox = ro_ref[:, 0:1]
    oy = ro_ref[:, 1:2]
    oz = ro_ref[:, 2:3]
    dx = px - ox
    dy = py - oy
    dz = pz - oz
    dnorm = jnp.sqrt((dx * dx + dy * dy) + dz * dz)
    maxd = jnp.maximum(dnorm, 1e-12)
    ux = dx / maxd
    uy = dy / maxd
    uz = dz / maxd
    lane = lax.broadcasted_iota(jnp.int32, (BF, NPAD), 1)
    colmask = lane < NPTS
    big = jnp.int32(2 ** 30)
    for r in range(NRAY):
        a = rd_ref[:, 3 * r:3 * r + 1]
        b = rd_ref[:, 3 * r + 1:3 * r + 2]
        c = rd_ref[:, 3 * r + 2:3 * r + 3]
        nrm = jnp.sqrt((a * a + b * b) + c * c)
        nrm = jnp.maximum(nrm, 1e-12)
        an = a / nrm
        bn = b / nrm
        cn = c / nrm
        cos = (an * ux + bn * uy) + cn * uz
        sinphi = jnp.sqrt(jnp.maximum(1.0 - cos * cos, 1e-12))
        proj = sinphi * dnorm
        proj = jnp.where(cos < 0.866, 1e8, proj)
        p = jnp.where(colmask, proj, jnp.inf)
        dvals = []
        ivals = []
        for k in range(KSEL):
            m = jnp.min(p, axis=1, keepdims=True)
            eqm = p == m
            im = jnp.min(jnp.where(eqm, lane, big), axis=1, keepdims=True)
            dvals.append(m)
            ivals.append(im)
            if k < KSEL - 1:
                p = jnp.where(lane == im, jnp.inf, p)
        dist_ref[:, r, :] = jnp.concatenate(dvals, axis=1)
        idx_ref[:, r, :] = jnp.concatenate(ivals, axis=1)


def _dist_topk(ray_o, ray_d24, pts_t):
    return pl.pallas_call(
        _dist_topk_body,
        grid=(NF // BF,),
        in_specs=[
            pl.BlockSpec((BF, 3), lambda i: (i, 0)),
            pl.BlockSpec((BF, 24), lambda i: (i, 0)),
            pl.BlockSpec((3, NPAD), lambda i: (0, 0)),
        ],
        out_specs=[
            pl.BlockSpec((BF, NRAY, KSEL), lambda i: (i, 0, 0)),
            pl.BlockSpec((BF, NRAY, KSEL), lambda i: (i, 0, 0)),
        ],
        out_shape=[
            jax.ShapeDtypeStruct((NF, NRAY, KSEL), jnp.float32),
            jax.ShapeDtypeStruct((NF, NRAY, KSEL), jnp.int32),
        ],
    )(ray_o, ray_d24, pts_t)


_NC = 2
_NS = 16
_NW = _NC * _NS
_BPW = NSEL // _NW


def _gather_body(pts_hbm, idx_hbm, out_hbm, idx_v, rows_v, sem):
    wid = lax.axis_index("s") * _NC + lax.axis_index("c")
    base = wid * _BPW
    pltpu.sync_copy(idx_hbm.at[pl.ds(base, _BPW)], idx_v)
    pltpu.async_copy(pts_hbm.at[idx_v], rows_v, sem).wait()
    pltpu.sync_copy(rows_v, out_hbm.at[pl.ds(base, _BPW)])


def _gather_sc(pts16, idx_flat):
    return pl.kernel(
        _gather_body,
        out_type=jax.ShapeDtypeStruct((NSEL, DPAD), jnp.float32),
        mesh=plsc.VectorSubcoreMesh(core_axis_name="c", subcore_axis_name="s"),
        scratch_types=[
            pltpu.VMEM((_BPW,), jnp.int32),
            pltpu.VMEM((_BPW, DPAD), jnp.float32),
            pltpu.SemaphoreType.DMA,
        ],
        compiler_params=pltpu.CompilerParams(use_tc_tiling_on_sc=False),
    )(pts16, idx_flat)


def _acos(x):
    a = jnp.abs(x)
    p = jnp.float32(-0.0012624911)
    for coef in (0.0066700901, -0.0170881256, 0.0308918810, -0.0501743046,
                 0.0889789874, -0.2145988016, 1.5707963050):
        p = p * a + jnp.float32(coef)
    r = jnp.sqrt(jnp.maximum(1.0 - a, 0.0)) * p
    return jnp.where(x < 0.0, jnp.float32(math.pi) - r, r)


def _feat_body(gx_ref, gy_ref, gz_ref, ro_ref, dist_ref,
               npd_ref, elev_ref, azim_ref, sky_ref):
    eps = 1e-5
    gx = gx_ref[...] - ro_ref[:, 0:1]
    gy = gy_ref[...] - ro_ref[:, 1:2]
    gz = gz_ref[...] - ro_ref[:, 2:3]
    npd = jnp.sqrt((gx * gx + gy * gy) + gz * gz)
    u1 = gz / (npd + eps)
    elev = _acos(u1)
    sin_elev = jnp.sqrt(jnp.maximum(1.0 - u1 * u1, 0.0))
    az = _acos(gx / (npd * sin_elev + eps))
    az = jnp.where(gy < 0.0, 2.0 * math.pi - az, az)
    npd_ref[...] = npd
    elev_ref[...] = elev
    azim_ref[...] = az
    sky_ref[...] = (dist_ref[...] >= 1e8 - 1).astype(jnp.int32)


def _features(gx, gy, gz, ray_o, dist):
    n = NRAY * KSEL
    return pl.pallas_call(
        _feat_body,
        grid=(1,),
        in_specs=[
            pl.BlockSpec((NF, n), lambda i: (0, 0)),
            pl.BlockSpec((NF, n), lambda i: (0, 0)),
            pl.BlockSpec((NF, n), lambda i: (0, 0)),
            pl.BlockSpec((NF, 3), lambda i: (0, 0)),
            pl.BlockSpec((NF, n), lambda i: (0, 0)),
        ],
        out_specs=[pl.BlockSpec((NF, n), lambda i: (0, 0))] * 4,
        out_shape=[
            jax.ShapeDtypeStruct((NF, n), jnp.float32),
            jax.ShapeDtypeStruct((NF, n), jnp.float32),
            jax.ShapeDtypeStruct((NF, n), jnp.float32),
            jax.ShapeDtypeStruct((NF, n), jnp.int32),
        ],
    )(gx, gy, gz, ray_o, dist)


def kernel(ray_o, ray_d, pts):
    ray_d24 = ray_d.reshape(NF, NRAY * 3)
    pts_t = jnp.pad(pts.T, ((0, 0), (0, NPAD - NPTS)))
    dist, idx = _dist_topk(ray_o, ray_d24, pts_t)

    pts16 = jnp.pad(pts, ((0, 0), (0, DPAD - 3)))
    rows = _gather_sc(pts16, idx.reshape(NSEL))

    g = rows[:, :3].reshape(NF, NRAY * KSEL, 3)
    npd, elev, azim, sky = _features(
        g[:, :, 0], g[:, :, 1], g[:, :, 2], ray_o,
        dist.reshape(NF, NRAY * KSEL))

    shp = (NF, NRAY, KSEL, 1)
    return (dist, idx, sky.astype(bool).reshape(NF, NRAY, KSEL),
            npd.reshape(shp), elev.reshape(shp), azim.reshape(shp))
